# baseline (speedup 1.0000x reference)
import functools

import numpy as np
import jax
import jax.numpy as jnp
from jax import lax
from jax.experimental import pallas as pl
from jax.experimental.pallas import tpu as pltpu
from jax.experimental.pallas import tpu_sc as plsc

_N_DOW, _N_MONTH, _N_LEAP, _N_DEC = 7, 12, 2, 16
_ED, _DIM, _B = 16, 64, 16384
_C = _N_DOW * _N_MONTH * _N_LEAP * _N_DEC


def _build_onehots():
    c = np.arange(_C)
    d = c // (_N_MONTH * _N_LEAP * _N_DEC)
    m = (c // (_N_LEAP * _N_DEC)) % _N_MONTH
    lp = (c // _N_DEC) % _N_LEAP
    dec = c % _N_DEC

    def oh(idx, n):
        a = np.zeros((_C, n), np.float32)
        a[np.arange(_C), idx] = 1.0
        return a

    return oh(d, _N_DOW), oh(m, _N_MONTH), oh(lp, _N_LEAP), oh(dec, _N_DEC)


_G_D, _G_M, _G_L, _G_DEC = _build_onehots()


def _table_body(gd, gm, gl, gdec, ed, em, el, edec, w1, b1, o_ref):
    f32 = jnp.float32
    h = jnp.dot(gd[...], jnp.dot(ed[...], w1[0:16, :], preferred_element_type=f32),
                preferred_element_type=f32)
    h += jnp.dot(gm[...], jnp.dot(em[...], w1[16:32, :], preferred_element_type=f32),
                 preferred_element_type=f32)
    h += jnp.dot(gl[...], jnp.dot(el[...], w1[32:48, :], preferred_element_type=f32),
                 preferred_element_type=f32)
    h += jnp.dot(gdec[...], jnp.dot(edec[...], w1[48:64, :], preferred_element_type=f32),
                 preferred_element_type=f32)
    h += b1[...]
    o_ref[:, 0:_DIM] = h


def _compute_table(dow_emb, month_emb, leap_emb, decade_emb, W1, b1):
    return pl.pallas_call(
        _table_body,
        out_shape=jax.ShapeDtypeStruct((_C, 2 * _DIM), jnp.float32),
    )(jnp.asarray(_G_D), jnp.asarray(_G_M), jnp.asarray(_G_L), jnp.asarray(_G_DEC),
      dow_emb, month_emb, leap_emb, decade_emb, W1, b1.reshape(1, _DIM))


def _epilogue_body(hw, w2, b2, o_ref):
    f32 = jnp.float32
    h = hw[:, 0:_DIM]
    h = 0.5 * h * (1.0 + lax.erf(h * np.float32(0.7071067811865476)))
    o_ref[...] = jnp.dot(h, w2[...], preferred_element_type=f32) + b2[...]


_EPI_BLK = 2048


def _epilogue(wide, W2, b2):
    nblk = _B // _EPI_BLK
    return pl.pallas_call(
        _epilogue_body,
        grid=(nblk,),
        in_specs=[
            pl.BlockSpec((_EPI_BLK, 2 * _DIM), lambda i: (i, 0)),
            pl.BlockSpec((_DIM, _DIM), lambda i: (0, 0)),
            pl.BlockSpec((1, _DIM), lambda i: (0, 0)),
        ],
        out_specs=pl.BlockSpec((_EPI_BLK, _DIM), lambda i: (i, 0)),
        out_shape=jax.ShapeDtypeStruct((_B, _DIM), jnp.float32),
    )(wide, W2, b2.reshape(1, _DIM))


@functools.lru_cache(maxsize=1)
def _make_gather():
    NC, NS, L = 2, 16, 16
    NW = NC * NS
    BPW = _B // NW
    NCHUNK = BPW // 128
    mesh = plsc.VectorSubcoreMesh(core_axis_name="c", subcore_axis_name="s")

    @functools.partial(
        pl.kernel, mesh=mesh,
        out_type=jax.ShapeDtypeStruct((_B, 2 * _DIM), jnp.float32),
        scratch_types=[
            pltpu.VMEM((4, BPW), jnp.int32),
            pltpu.VMEM((BPW,), jnp.int32),
            pltpu.VMEM((BPW, 2 * _DIM), jnp.float32),
            pltpu.SemaphoreType.DMA,
            pltpu.SemaphoreType.DMA,
            pltpu.SemaphoreType.DMA,
            pltpu.SemaphoreType.DMA,
            pltpu.SemaphoreType.DMA,
            pltpu.SemaphoreType.DMA,
        ],
    )
    def gather(table_hbm, dow_hbm, month_hbm, leap_hbm, dec_hbm, out_hbm,
               in_v, idx_v, rows_v, sem, g0, g1, g2, g3, sem_w):
        sem_g = (g0, g1, g2, g3)
        wid = lax.axis_index("s") * NC + lax.axis_index("c")
        base = wid * BPW
        loads = [
            pltpu.async_copy(src.at[pl.ds(base, BPW)], in_v.at[k], sem)
            for k, src in enumerate((dow_hbm, month_hbm, leap_hbm, dec_hbm))
        ]
        for cp in loads:
            cp.wait()

        def _combine(i, _):
            s = pl.ds(i * L, L)
            idx_v[s] = ((in_v[0, s] * 12 + in_v[1, s]) * 2
                        + in_v[2, s]) * 16 + in_v[3, s]
            return ()

        lax.fori_loop(0, BPW // L, _combine, ())
        gathers = [
            pltpu.async_copy(table_hbm.at[idx_v.at[pl.ds(j * 128, 128)]],
                             rows_v.at[pl.ds(j * 128, 128)], sem_g[j])
            for j in range(NCHUNK)
        ]
        writes = []
        for j in range(NCHUNK):
            gathers[j].wait()
            writes.append(
                pltpu.async_copy(rows_v.at[pl.ds(j * 128, 128)],
                                 out_hbm.at[pl.ds(base + j * 128, 128)], sem_w))
        for cp in writes:
            cp.wait()

    return gather


def kernel(dow, month, leap, decade, dow_emb, month_emb, leap_emb, decade_emb,
           W1, b1, W2, b2):
    table = _compute_table(dow_emb, month_emb, leap_emb, decade_emb, W1, b1)
    wide = _make_gather()(table, dow.astype(jnp.int32), month.astype(jnp.int32),
                          leap.astype(jnp.int32), decade.astype(jnp.int32))
    return _epilogue(wide, W2, b2)


# speedup vs baseline: 1.3523x; 1.3523x over previous
import functools

import numpy as np
import jax
import jax.numpy as jnp
from jax import lax
from jax.experimental import pallas as pl
from jax.experimental.pallas import tpu as pltpu
from jax.experimental.pallas import tpu_sc as plsc

_N_DOW, _N_MONTH, _N_LEAP, _N_DEC = 7, 12, 2, 16
_ED, _DIM, _B = 16, 64, 16384
_C = _N_DOW * _N_MONTH * _N_LEAP * _N_DEC


def _build_onehots():
    c = np.arange(_C)
    d = c // (_N_MONTH * _N_LEAP * _N_DEC)
    m = (c // (_N_LEAP * _N_DEC)) % _N_MONTH
    lp = (c // _N_DEC) % _N_LEAP
    dec = c % _N_DEC

    a = np.zeros((_C, _DIM), np.float32)
    r = np.arange(_C)
    a[r, d] = 1.0
    a[r, 7 + m] = 1.0
    a[r, 19 + lp] = 1.0
    a[r, 21 + dec] = 1.0
    return a


_G4 = _build_onehots()


def _table_body(g4, ed, em, el, edec, w1, b1, w2, b2, o_ref):
    f32 = jnp.float32
    a = jnp.concatenate([
        jnp.dot(ed[...], w1[0:16, :], preferred_element_type=f32),
        jnp.dot(em[...], w1[16:32, :], preferred_element_type=f32),
        jnp.dot(el[...], w1[32:48, :], preferred_element_type=f32),
        jnp.dot(edec[...], w1[48:64, :], preferred_element_type=f32),
        jnp.zeros((27, _DIM), f32),
    ], axis=0)
    h = jnp.dot(g4[...], a, preferred_element_type=f32) + b1[...]
    h = 0.5 * h * (1.0 + lax.erf(h * np.float32(0.7071067811865476)))
    o = jnp.dot(h, w2[...], preferred_element_type=f32) + b2[...]
    o_ref[:, 0:_DIM] = o


def _compute_table(dow_emb, month_emb, leap_emb, decade_emb, W1, b1, W2, b2):
    return pl.pallas_call(
        _table_body,
        out_shape=jax.ShapeDtypeStruct((_C, 2 * _DIM), jnp.float32),
    )(jnp.asarray(_G4), dow_emb, month_emb, leap_emb, decade_emb,
      W1, b1.reshape(1, _DIM), W2, b2.reshape(1, _DIM))


@functools.lru_cache(maxsize=1)
def _make_gather():
    NC, NS, L = 2, 16, 16
    NW = NC * NS
    BPW = _B // NW
    NCHUNK = BPW // 128
    mesh = plsc.VectorSubcoreMesh(core_axis_name="c", subcore_axis_name="s")

    @functools.partial(
        pl.kernel, mesh=mesh,
        out_type=jax.ShapeDtypeStruct((_B, 2 * _DIM), jnp.float32),
        scratch_types=[
            pltpu.VMEM((4, BPW), jnp.int32),
            pltpu.VMEM((BPW,), jnp.int32),
            pltpu.VMEM((BPW, 2 * _DIM), jnp.float32),
            pltpu.SemaphoreType.DMA,
            pltpu.SemaphoreType.DMA,
            pltpu.SemaphoreType.DMA,
            pltpu.SemaphoreType.DMA,
            pltpu.SemaphoreType.DMA,
            pltpu.SemaphoreType.DMA,
        ],
    )
    def gather(table_hbm, dow_hbm, month_hbm, leap_hbm, dec_hbm, out_hbm,
               in_v, idx_v, rows_v, sem, g0, g1, g2, g3, sem_w):
        sem_g = (g0, g1, g2, g3)
        wid = lax.axis_index("s") * NC + lax.axis_index("c")
        base = wid * BPW
        loads = [
            pltpu.async_copy(src.at[pl.ds(base, BPW)], in_v.at[k], sem)
            for k, src in enumerate((dow_hbm, month_hbm, leap_hbm, dec_hbm))
        ]
        for cp in loads:
            cp.wait()

        def _combine(i, _):
            s = pl.ds(i * L, L)
            idx_v[s] = ((in_v[0, s] * 12 + in_v[1, s]) * 2
                        + in_v[2, s]) * 16 + in_v[3, s]
            return ()

        lax.fori_loop(0, BPW // L, _combine, ())
        gathers = [
            pltpu.async_copy(table_hbm.at[idx_v.at[pl.ds(j * 128, 128)]],
                             rows_v.at[pl.ds(j * 128, 128)], sem_g[j])
            for j in range(NCHUNK)
        ]
        writes = []
        for j in range(NCHUNK):
            gathers[j].wait()
            writes.append(
                pltpu.async_copy(rows_v.at[pl.ds(j * 128, 128)],
                                 out_hbm.at[pl.ds(base + j * 128, 128)], sem_w))
        for cp in writes:
            cp.wait()

    return gather


def kernel(dow, month, leap, decade, dow_emb, month_emb, leap_emb, decade_emb,
           W1, b1, W2, b2):
    table = _compute_table(dow_emb, month_emb, leap_emb, decade_emb, W1, b1, W2, b2)
    wide = _make_gather()(table, dow.astype(jnp.int32), month.astype(jnp.int32),
                          leap.astype(jnp.int32), decade.astype(jnp.int32))
    return wide[:, :_DIM]
